# baseline (speedup 1.0000x reference)
import functools

import jax
import jax.numpy as jnp
from jax import lax
from jax.experimental import pallas as pl
from jax.experimental.pallas import tpu as pltpu
from jax.experimental.pallas import tpu_sc as plsc

B = 4
L = 512
V = 10000
VP = 10240
NNZ = 160000
VOCAB = 30522
EMB = 768
HID = 128
OUT = 128

NT = 32
EPT = NNZ // NT
REGC = 5760
R3 = 3 * REGC
PBUF = 5024
CHE = 640
DPT = OUT // NT
RPT = VP // NT
CG = 64

_mesh = plsc.VectorSubcoreMesh(core_axis_name="c", subcore_axis_name="s")
_sc_params = pltpu.CompilerParams(needs_layout_passes=False)


def _wid():
    return lax.axis_index("c") * 16 + lax.axis_index("s")


def _w2t_body(fcw_ref, wv_ref, out_ref):
    out_ref[...] = lax.dot_general(
        fcw_ref[...], wv_ref[...],
        (((1,), (1,)), ((), ())),
        preferred_element_type=jnp.float32)


def _w2t_call(fc_w, wv_pad):
    return pl.pallas_call(
        _w2t_body,
        out_shape=jax.ShapeDtypeStruct((OUT, VP), jnp.float32),
    )(fc_w, wv_pad)


def _compact_body(ids_hbm, tok_hbm, row_hbm, col_hbm, val_hbm,
                  edges_hbm, cnt_hbm,
                  tok_v, ids_v, inset_v, er_v, ec_v, ev_v,
                  rb_a, cb_a, vb_a, rb_b, cb_b, vb_b, cnt_v, sem_w):
    w = _wid()
    iota16 = jnp.arange(16, dtype=jnp.int32)
    zeros16 = jnp.zeros((16,), jnp.int32)
    ones16 = jnp.ones((16,), jnp.int32)

    pltpu.sync_copy(tok_hbm, tok_v)
    pltpu.sync_copy(ids_hbm, ids_v)
    pltpu.sync_copy(row_hbm.at[pl.ds(w * EPT, EPT)], er_v.at[pl.ds(0, EPT)])
    pltpu.sync_copy(col_hbm.at[pl.ds(w * EPT, EPT)], ec_v.at[pl.ds(0, EPT)])
    pltpu.sync_copy(val_hbm.at[pl.ds(w * EPT, EPT)], ev_v.at[pl.ds(0, EPT)])
    er_v[pl.ds(EPT, 16)] = zeros16
    ec_v[pl.ds(EPT, 16)] = zeros16
    ev_v[pl.ds(EPT, 16)] = zeros16

    def zero_step(i, _):
        inset_v[pl.ds(i * 16, 16)] = zeros16
        return 0
    lax.fori_loop(0, (B * V) // 16, zero_step, 0)

    def build_step(i, _):
        b = i // (L // 16)
        s = i % (L // 16)
        idv = ids_v[pl.ds(b * L + s * 16, 16)]
        idv = jnp.where(idv > VOCAB - 1, 0, idv)
        g = plsc.load_gather(tok_v, [idv])
        plsc.store_scatter(inset_v, [g + b * V], ones16)
        return 0
    for i in range(B * (L // 16)):
        build_step(i, 0)

    nsteps = (EPT + 15) // 16
    cvec = zeros16
    for b in range(B):
        bufs = (rb_a, cb_a, vb_a) if b % 2 == 0 else (rb_b, cb_b, vb_b)
        rb_v, cb_v, vb_v = bufs

        def comp_step(i, off, b=b, rb_v=rb_v, cb_v=cb_v, vb_v=vb_v):
            base = i * 16
            valid = (base + iota16) < EPT
            r = er_v[pl.ds(base, 16)]
            c = ec_v[pl.ds(base, 16)]
            v = ev_v[pl.ds(base, 16)]
            inr = plsc.load_gather(inset_v, [r + b * V])
            inc = plsc.load_gather(inset_v, [c + b * V])
            m = ((inr + inc) > 0) & valid
            plsc.store_compressed(rb_v.at[pl.ds(off, 16)], r, mask=m)
            plsc.store_compressed(cb_v.at[pl.ds(off, 16)], c, mask=m)
            plsc.store_compressed(vb_v.at[pl.ds(off, 16)], v, mask=m)
            pc = plsc.all_reduce_population_count(m)
            return off + pc[0]
        if b >= 2:
            for pb in bufs:
                pltpu.make_async_copy(pb, edges_hbm.at[pl.ds(0, PBUF)],
                                      sem_w).wait()
        off = lax.fori_loop(0, nsteps, comp_step, jnp.int32(0))
        cvec = jnp.where(iota16 == b, off, cvec)
        rbase = (b * NT + w) * R3
        for p, pb in enumerate(bufs):
            roff = pl.multiple_of(rbase + p * REGC, 8)
            pltpu.async_copy(pb, edges_hbm.at[pl.ds(roff, PBUF)], sem_w)
    cnt_v[...] = cvec
    for pb in (rb_a, cb_a, vb_a, rb_b, cb_b, vb_b):
        pltpu.make_async_copy(pb, edges_hbm.at[pl.ds(0, PBUF)], sem_w).wait()
    pltpu.sync_copy(cnt_v, cnt_hbm.at[pl.ds(w * 16, 16)])


def _compact_call(ids, tok, rows, cols, vals):
    f = pl.kernel(
        _compact_body,
        mesh=_mesh,
        compiler_params=_sc_params,
        out_type=(
            jax.ShapeDtypeStruct((B * NT * R3,), jnp.int32),
            jax.ShapeDtypeStruct((NT * 16,), jnp.int32),
        ),
        scratch_types=[
            pltpu.VMEM((VOCAB,), jnp.int32),
            pltpu.VMEM((B * L,), jnp.int32),
            pltpu.VMEM((B * V,), jnp.int32),
            pltpu.VMEM((EPT + 16,), jnp.int32),
            pltpu.VMEM((EPT + 16,), jnp.int32),
            pltpu.VMEM((EPT + 16,), jnp.int32),
            pltpu.VMEM((PBUF,), jnp.int32),
            pltpu.VMEM((PBUF,), jnp.int32),
            pltpu.VMEM((PBUF,), jnp.int32),
            pltpu.VMEM((PBUF,), jnp.int32),
            pltpu.VMEM((PBUF,), jnp.int32),
            pltpu.VMEM((PBUF,), jnp.int32),
            pltpu.VMEM((16,), jnp.int32),
            pltpu.SemaphoreType.DMA,
        ],
    )
    return f(ids, tok, rows, cols, vals)


def _gemb_body(gv_hbm, emb_hbm, gemb_hbm, gvi_v, rows_a, rows_b,
               sem_a, sem_b, sem_sa, sem_sb):
    w = _wid()
    nchunks = RPT // CG
    bufs = (rows_a, rows_b)
    sems = (sem_a, sem_b)
    ssems = (sem_sa, sem_sb)
    pltpu.sync_copy(gv_hbm.at[pl.ds(w * RPT, RPT)], gvi_v)
    gets = [None] * nchunks
    puts = [None] * nchunks
    gets[0] = pltpu.async_copy(
        emb_hbm.at[gvi_v.at[pl.ds(0, CG)]], bufs[0], sems[0])
    for c in range(nchunks):
        if c + 1 < nchunks:
            if c - 1 >= 0:
                puts[c - 1].wait()
            gets[c + 1] = pltpu.async_copy(
                emb_hbm.at[gvi_v.at[pl.ds((c + 1) * CG, CG)]],
                bufs[(c + 1) % 2], sems[(c + 1) % 2])
        gets[c].wait()
        puts[c] = pltpu.async_copy(
            bufs[c % 2],
            gemb_hbm.at[pl.ds(pl.multiple_of(w * RPT + c * CG, 8), CG)],
            ssems[c % 2])
    puts[nchunks - 2].wait()
    puts[nchunks - 1].wait()


def _gemb_call(gv_pad, emb_weight):
    f = pl.kernel(
        _gemb_body,
        mesh=_mesh,
        compiler_params=_sc_params,
        out_type=jax.ShapeDtypeStruct((VP, EMB), jnp.float32),
        scratch_types=[
            pltpu.VMEM((RPT,), jnp.int32),
            pltpu.VMEM((CG, EMB), jnp.float32),
            pltpu.VMEM((CG, EMB), jnp.float32),
            pltpu.SemaphoreType.DMA,
            pltpu.SemaphoreType.DMA,
            pltpu.SemaphoreType.DMA,
            pltpu.SemaphoreType.DMA,
        ],
    )
    return f(gv_pad, emb_weight)


def _scatter_body(w2t_hbm, edges_hbm, cnt_hbm,
                  gt_hbm, w2_v, g_v, cnt_v,
                  rb_a, cb_a, vb_a, rb_b, cb_b, vb_b, rb_c, cb_c, vb_c,
                  cnt_s, sem_a, sem_b):
    w = _wid()
    d0 = w * DPT
    iota16 = jnp.arange(16, dtype=jnp.int32)
    zeros16f = jnp.zeros((16,), jnp.float32)
    bufs_a = (rb_a, cb_a, vb_a)
    bufs_b = (rb_b, cb_b, vb_b)
    bufs_c = (rb_c, cb_c, vb_c)

    pltpu.sync_copy(w2t_hbm.at[pl.ds(pl.multiple_of(d0 * VP, 512), DPT * VP)],
                    w2_v)
    pltpu.sync_copy(cnt_hbm, cnt_v)

    for w2 in range(NT):
        cv = cnt_v[pl.ds(w2 * 16, 16)]
        for b in range(B):
            cnt_s[b * NT + w2] = cv[b]

    def issue(b, rg, bufs, sem, ck=0):
        rg = jnp.minimum(rg, NT - 1)
        rbase = (b * NT + rg) * R3 + ck
        for p in range(3):
            eoff = pl.multiple_of(rbase + p * REGC, 8)
            pltpu.async_copy(edges_hbm.at[pl.ds(eoff, CHE)], bufs[p], sem)

    def drain(bufs, sem):
        for p in range(3):
            pltpu.make_async_copy(edges_hbm.at[pl.ds(0, CHE)], bufs[p],
                                  sem).wait()

    def run_edges(bufs, lim):
        rbuf, cbuf, vbuf = bufs

        def group(base):
            r = rbuf[pl.ds(base, 16)]
            cc = cbuf[pl.ds(base, 16)]
            v = plsc.bitcast(vbuf[pl.ds(base, 16)], jnp.float32)
            for j in range(DPT):
                wv = plsc.load_gather(w2_v, [cc + j * VP])
                plsc.addupdate_scatter(g_v, [r + j * VP], wv * v)

        n64 = lim // 64

        def qstep(s, _):
            t = s * 64
            group(t)
            group(t + 16)
            group(t + 32)
            group(t + 48)
            return 0
        lax.fori_loop(0, n64, qstep, 0)

        n16 = (lim - n64 * 64) // 16
        tb0 = n64 * 64
        for u in range(3):
            @pl.when(n16 >= u + 1)
            def _(u=u):
                group(tb0 + u * 16)

        rem = lim - (lim // 16) * 16

        @pl.when(rem > 0)
        def _():
            base = (lim // 16) * 16
            valid = iota16 < rem
            r = jnp.where(valid, rbuf[pl.ds(base, 16)], 0)
            cc = jnp.where(valid, cbuf[pl.ds(base, 16)], 0)
            v = jnp.where(valid,
                          plsc.bitcast(vbuf[pl.ds(base, 16)], jnp.float32),
                          0.0)
            for j in range(DPT):
                wv = plsc.load_gather(w2_v, [cc + j * VP])
                plsc.addupdate_scatter(g_v, [r + j * VP], wv * v)

    def process(b, rg, bufs):
        n = cnt_s[b * NT + rg]
        run_edges(bufs, jnp.minimum(n, CHE))

        def cond(ck):
            return ck < n

        def obody(ck):
            rbase = (b * NT + rg) * R3 + ck
            for p in range(3):
                eoff = pl.multiple_of(rbase + p * REGC, 8)
                pltpu.sync_copy(edges_hbm.at[pl.ds(eoff, CHE)], bufs_c[p])
            run_edges(bufs_c, jnp.minimum(n - ck, CHE))
            return ck + CHE

        lax.while_loop(cond, obody, jnp.int32(CHE))

    def per_batch(b, _):
        def zero_step(i, _):
            for u in range(8):
                g_v[pl.ds(i * 128 + u * 16, 16)] = zeros16f
            return 0
        lax.fori_loop(0, (DPT * VP) // 128, zero_step, 0)

        issue(b, 0, bufs_a, sem_a)
        issue(b, 1, bufs_b, sem_b)

        def pair(p, _):
            drain(bufs_a, sem_a)
            process(b, 2 * p, bufs_a)
            issue(b, 2 * p + 2, bufs_a, sem_a)
            drain(bufs_b, sem_b)
            process(b, 2 * p + 1, bufs_b)
            issue(b, 2 * p + 3, bufs_b, sem_b)
            return 0

        lax.fori_loop(0, NT // 2, pair, 0)
        drain(bufs_a, sem_a)
        drain(bufs_b, sem_b)
        goff = pl.multiple_of(b * OUT * VP + d0 * VP, 512)
        pltpu.sync_copy(g_v, gt_hbm.at[pl.ds(goff, DPT * VP)])
        return 0

    lax.fori_loop(0, B, per_batch, 0)


def _scatter_call(w2t_flat, edges, counts):
    f = pl.kernel(
        _scatter_body,
        mesh=_mesh,
        compiler_params=_sc_params,
        out_type=jax.ShapeDtypeStruct((B * OUT * VP,), jnp.float32),
        scratch_types=[
            pltpu.VMEM((DPT * VP,), jnp.float32),
            pltpu.VMEM((DPT * VP,), jnp.float32),
            pltpu.VMEM((NT * 16,), jnp.int32),
            pltpu.VMEM((CHE,), jnp.int32),
            pltpu.VMEM((CHE,), jnp.int32),
            pltpu.VMEM((CHE,), jnp.int32),
            pltpu.VMEM((CHE,), jnp.int32),
            pltpu.VMEM((CHE,), jnp.int32),
            pltpu.VMEM((CHE,), jnp.int32),
            pltpu.VMEM((CHE,), jnp.int32),
            pltpu.VMEM((CHE,), jnp.int32),
            pltpu.VMEM((CHE,), jnp.int32),
            pltpu.SMEM((B * NT,), jnp.int32),
            pltpu.SemaphoreType.DMA,
            pltpu.SemaphoreType.DMA,
        ],
    )
    return f(w2t_flat, edges, counts)


KBLK = 1280


def _mm_body(a_ref, b_ref, bias_ref, o_ref):
    k = pl.program_id(0)
    acc = jnp.dot(a_ref[...].astype(jnp.bfloat16),
                  b_ref[...].astype(jnp.bfloat16),
                  preferred_element_type=jnp.float32)

    @pl.when(k == 0)
    def _():
        o_ref[...] = acc + bias_ref[...]

    @pl.when(k != 0)
    def _():
        o_ref[...] += acc


def _mm_call(g2, gemb, bias):
    return pl.pallas_call(
        _mm_body,
        grid=(VP // KBLK,),
        in_specs=[
            pl.BlockSpec((B * OUT, KBLK), lambda k: (0, k)),
            pl.BlockSpec((KBLK, EMB), lambda k: (k, 0)),
            pl.BlockSpec((B * OUT, EMB), lambda k: (0, 0)),
        ],
        out_specs=pl.BlockSpec((B * OUT, EMB), lambda k: (0, 0)),
        out_shape=jax.ShapeDtypeStruct((B * OUT, EMB), jnp.float32),
        compiler_params=pltpu.CompilerParams(
            dimension_semantics=("arbitrary",)),
    )(g2, gemb, bias)


@jax.jit
def kernel(input_ids, emb_weight, adj_indices, adj_values, W_vh, tok_to_graph,
           gv_ids, fc_w, fc_b):
    ids = input_ids.astype(jnp.int32).reshape(-1)
    tok = tok_to_graph.astype(jnp.int32)
    rows = adj_indices[0].astype(jnp.int32)
    cols = adj_indices[1].astype(jnp.int32)
    vals = lax.bitcast_convert_type(adj_values.astype(jnp.float32), jnp.int32)
    wv_pad = jnp.pad(W_vh.astype(jnp.float32), ((0, VP - V), (0, 0)))
    gv_pad = jnp.pad(gv_ids.astype(jnp.int32), (0, VP - V))

    w2t = _w2t_call(fc_w.astype(jnp.float32), wv_pad)
    edges, counts = _compact_call(ids, tok, rows, cols, vals)
    gemb = _gemb_call(gv_pad, emb_weight.astype(jnp.float32))
    gt = _scatter_call(w2t.reshape(-1), edges, counts)

    g2 = gt.reshape(B * OUT, VP)
    bias = jnp.broadcast_to(jnp.tile(fc_b.astype(jnp.float32), B)[:, None],
                            (B * OUT, EMB))
    out = _mm_call(g2, gemb, bias)
    return out.reshape(B, OUT, EMB)


# speedup vs baseline: 1.0313x; 1.0313x over previous
import functools

import jax
import jax.numpy as jnp
from jax import lax
from jax.experimental import pallas as pl
from jax.experimental.pallas import tpu as pltpu
from jax.experimental.pallas import tpu_sc as plsc

B = 4
L = 512
V = 10000
VP = 10240
NNZ = 160000
VOCAB = 30522
EMB = 768
HID = 128
OUT = 128

NT = 32
EPT = NNZ // NT
REGC = 5760
R3 = 3 * REGC
PBUF = 5024
CHE = 640
DPT = OUT // NT
RPT = VP // NT
CG = 16
NCH = RPT // CG

_mesh = plsc.VectorSubcoreMesh(core_axis_name="c", subcore_axis_name="s")
_sc_params = pltpu.CompilerParams(needs_layout_passes=False)


def _wid():
    return lax.axis_index("c") * 16 + lax.axis_index("s")


def _w2t_body(fcw_ref, wv_ref, out_ref):
    out_ref[...] = lax.dot_general(
        fcw_ref[...], wv_ref[...],
        (((1,), (1,)), ((), ())),
        preferred_element_type=jnp.float32)


def _w2t_call(fc_w, wv_pad):
    return pl.pallas_call(
        _w2t_body,
        out_shape=jax.ShapeDtypeStruct((OUT, VP), jnp.float32),
    )(fc_w, wv_pad)


def _compact_body(ids_hbm, tok_hbm, row_hbm, col_hbm, val_hbm,
                  edges_hbm, cnt_hbm,
                  tok_v, ids_v, inset_v, er_v, ec_v, ev_v,
                  rb_a, cb_a, vb_a, rb_b, cb_b, vb_b, cnt_v, sem_w):
    w = _wid()
    iota16 = jnp.arange(16, dtype=jnp.int32)
    zeros16 = jnp.zeros((16,), jnp.int32)
    ones16 = jnp.ones((16,), jnp.int32)

    pltpu.sync_copy(tok_hbm, tok_v)
    pltpu.sync_copy(ids_hbm, ids_v)
    pltpu.sync_copy(row_hbm.at[pl.ds(w * EPT, EPT)], er_v.at[pl.ds(0, EPT)])
    pltpu.sync_copy(col_hbm.at[pl.ds(w * EPT, EPT)], ec_v.at[pl.ds(0, EPT)])
    pltpu.sync_copy(val_hbm.at[pl.ds(w * EPT, EPT)], ev_v.at[pl.ds(0, EPT)])
    er_v[pl.ds(EPT, 16)] = zeros16
    ec_v[pl.ds(EPT, 16)] = zeros16
    ev_v[pl.ds(EPT, 16)] = zeros16

    def zero_step(i, _):
        inset_v[pl.ds(i * 16, 16)] = zeros16
        return 0
    lax.fori_loop(0, (B * V) // 16, zero_step, 0)

    def build_step(i, _):
        b = i // (L // 16)
        s = i % (L // 16)
        idv = ids_v[pl.ds(b * L + s * 16, 16)]
        idv = jnp.where(idv > VOCAB - 1, 0, idv)
        g = plsc.load_gather(tok_v, [idv])
        plsc.store_scatter(inset_v, [g + b * V], ones16)
        return 0
    for i in range(B * (L // 16)):
        build_step(i, 0)

    nsteps = (EPT + 15) // 16
    cvec = zeros16
    for b in range(B):
        bufs = (rb_a, cb_a, vb_a) if b % 2 == 0 else (rb_b, cb_b, vb_b)
        rb_v, cb_v, vb_v = bufs

        def comp_step(i, off, b=b, rb_v=rb_v, cb_v=cb_v, vb_v=vb_v):
            base = i * 16
            valid = (base + iota16) < EPT
            r = er_v[pl.ds(base, 16)]
            c = ec_v[pl.ds(base, 16)]
            v = ev_v[pl.ds(base, 16)]
            inr = plsc.load_gather(inset_v, [r + b * V])
            inc = plsc.load_gather(inset_v, [c + b * V])
            m = ((inr + inc) > 0) & valid
            plsc.store_compressed(rb_v.at[pl.ds(off, 16)], r, mask=m)
            plsc.store_compressed(cb_v.at[pl.ds(off, 16)], c, mask=m)
            plsc.store_compressed(vb_v.at[pl.ds(off, 16)], v, mask=m)
            pc = plsc.all_reduce_population_count(m)
            return off + pc[0]
        if b >= 2:
            for pb in bufs:
                pltpu.make_async_copy(pb, edges_hbm.at[pl.ds(0, PBUF)],
                                      sem_w).wait()
        off = lax.fori_loop(0, nsteps, comp_step, jnp.int32(0))
        cvec = jnp.where(iota16 == b, off, cvec)
        rbase = (b * NT + w) * R3
        for p, pb in enumerate(bufs):
            roff = pl.multiple_of(rbase + p * REGC, 8)
            pltpu.async_copy(pb, edges_hbm.at[pl.ds(roff, PBUF)], sem_w)
    cnt_v[...] = cvec
    for pb in (rb_a, cb_a, vb_a, rb_b, cb_b, vb_b):
        pltpu.make_async_copy(pb, edges_hbm.at[pl.ds(0, PBUF)], sem_w).wait()
    pltpu.sync_copy(cnt_v, cnt_hbm.at[pl.ds(w * 16, 16)])


def _compact_call(ids, tok, rows, cols, vals):
    f = pl.kernel(
        _compact_body,
        mesh=_mesh,
        compiler_params=_sc_params,
        out_type=(
            jax.ShapeDtypeStruct((B * NT * R3,), jnp.int32),
            jax.ShapeDtypeStruct((NT * 16,), jnp.int32),
        ),
        scratch_types=[
            pltpu.VMEM((VOCAB,), jnp.int32),
            pltpu.VMEM((B * L,), jnp.int32),
            pltpu.VMEM((B * V,), jnp.int32),
            pltpu.VMEM((EPT + 16,), jnp.int32),
            pltpu.VMEM((EPT + 16,), jnp.int32),
            pltpu.VMEM((EPT + 16,), jnp.int32),
            pltpu.VMEM((PBUF,), jnp.int32),
            pltpu.VMEM((PBUF,), jnp.int32),
            pltpu.VMEM((PBUF,), jnp.int32),
            pltpu.VMEM((PBUF,), jnp.int32),
            pltpu.VMEM((PBUF,), jnp.int32),
            pltpu.VMEM((PBUF,), jnp.int32),
            pltpu.VMEM((16,), jnp.int32),
            pltpu.SemaphoreType.DMA,
        ],
    )
    return f(ids, tok, rows, cols, vals)


def _scatter_body(w2t_hbm, edges_hbm, cnt_hbm, gv_hbm, emb_hbm,
                  gt_hbm, gemb_hbm, w2_v, g_v, cnt_v,
                  rb_a, cb_a, vb_a, rb_b, cb_b, vb_b, rb_c, cb_c, vb_c,
                  gvi_v, gb_a, gb_b,
                  cnt_s, sem_a, sem_b, gsem_a, gsem_b):
    w = _wid()
    d0 = w * DPT
    iota16 = jnp.arange(16, dtype=jnp.int32)
    zeros16f = jnp.zeros((16,), jnp.float32)
    bufs_a = (rb_a, cb_a, vb_a)
    bufs_b = (rb_b, cb_b, vb_b)
    bufs_c = (rb_c, cb_c, vb_c)

    pltpu.sync_copy(w2t_hbm.at[pl.ds(pl.multiple_of(d0 * VP, 512), DPT * VP)],
                    w2_v)
    pltpu.sync_copy(cnt_hbm, cnt_v)
    pltpu.sync_copy(gv_hbm.at[pl.ds(w * RPT, RPT)], gvi_v)

    def gemb_issue(ch, buf, sem):
        goff = pl.multiple_of(ch * CG, 8)
        pltpu.async_copy(emb_hbm.at[gvi_v.at[pl.ds(goff, CG)]], buf, sem)

    def gemb_drain(buf, sem):
        pltpu.make_async_copy(emb_hbm.at[pl.ds(0, CG)], buf, sem).wait()

    def gemb_store(ch, buf):
        roff = pl.multiple_of(w * RPT + ch * CG, 8)
        pltpu.sync_copy(buf, gemb_hbm.at[pl.ds(roff, CG)])

    def gemb_service(q):
        @pl.when((q >= 1) & (q <= NCH // 2))
        def _():
            gemb_drain(gb_a, gsem_a)
            gemb_store(2 * (q - 1), gb_a)
            gemb_drain(gb_b, gsem_b)
            gemb_store(2 * q - 1, gb_b)

        @pl.when(q <= NCH // 2 - 1)
        def _():
            gemb_issue(2 * q, gb_a, gsem_a)
            gemb_issue(2 * q + 1, gb_b, gsem_b)

    for w2 in range(NT):
        cv = cnt_v[pl.ds(w2 * 16, 16)]
        for b in range(B):
            cnt_s[b * NT + w2] = cv[b]

    def issue(b, rg, bufs, sem, ck=0):
        rg = jnp.minimum(rg, NT - 1)
        rbase = (b * NT + rg) * R3 + ck
        for p in range(3):
            eoff = pl.multiple_of(rbase + p * REGC, 8)
            pltpu.async_copy(edges_hbm.at[pl.ds(eoff, CHE)], bufs[p], sem)

    def drain(bufs, sem):
        for p in range(3):
            pltpu.make_async_copy(edges_hbm.at[pl.ds(0, CHE)], bufs[p],
                                  sem).wait()

    def run_edges(bufs, lim):
        rbuf, cbuf, vbuf = bufs

        def group(base):
            r = rbuf[pl.ds(base, 16)]
            cc = cbuf[pl.ds(base, 16)]
            v = plsc.bitcast(vbuf[pl.ds(base, 16)], jnp.float32)
            for j in range(DPT):
                wv = plsc.load_gather(w2_v, [cc + j * VP])
                plsc.addupdate_scatter(g_v, [r + j * VP], wv * v)

        n64 = lim // 64

        def qstep(s, _):
            t = s * 64
            group(t)
            group(t + 16)
            group(t + 32)
            group(t + 48)
            return 0
        lax.fori_loop(0, n64, qstep, 0)

        n16 = (lim - n64 * 64) // 16
        tb0 = n64 * 64
        for u in range(3):
            @pl.when(n16 >= u + 1)
            def _(u=u):
                group(tb0 + u * 16)

        rem = lim - (lim // 16) * 16

        @pl.when(rem > 0)
        def _():
            base = (lim // 16) * 16
            valid = iota16 < rem
            r = jnp.where(valid, rbuf[pl.ds(base, 16)], 0)
            cc = jnp.where(valid, cbuf[pl.ds(base, 16)], 0)
            v = jnp.where(valid,
                          plsc.bitcast(vbuf[pl.ds(base, 16)], jnp.float32),
                          0.0)
            for j in range(DPT):
                wv = plsc.load_gather(w2_v, [cc + j * VP])
                plsc.addupdate_scatter(g_v, [r + j * VP], wv * v)

    def process(b, rg, bufs):
        n = cnt_s[b * NT + rg]
        run_edges(bufs, jnp.minimum(n, CHE))

        def cond(ck):
            return ck < n

        def obody(ck):
            rbase = (b * NT + rg) * R3 + ck
            for p in range(3):
                eoff = pl.multiple_of(rbase + p * REGC, 8)
                pltpu.sync_copy(edges_hbm.at[pl.ds(eoff, CHE)], bufs_c[p])
            run_edges(bufs_c, jnp.minimum(n - ck, CHE))
            return ck + CHE

        lax.while_loop(cond, obody, jnp.int32(CHE))

    def per_batch(b, _):
        def zero_step(i, _):
            for u in range(8):
                g_v[pl.ds(i * 128 + u * 16, 16)] = zeros16f
            return 0
        lax.fori_loop(0, (DPT * VP) // 128, zero_step, 0)

        issue(b, 0, bufs_a, sem_a)
        issue(b, 1, bufs_b, sem_b)

        def pair(p, _):
            gemb_service(b * (NT // 2) + p)
            drain(bufs_a, sem_a)
            process(b, 2 * p, bufs_a)
            issue(b, 2 * p + 2, bufs_a, sem_a)
            drain(bufs_b, sem_b)
            process(b, 2 * p + 1, bufs_b)
            issue(b, 2 * p + 3, bufs_b, sem_b)
            return 0

        lax.fori_loop(0, NT // 2, pair, 0)
        drain(bufs_a, sem_a)
        drain(bufs_b, sem_b)
        goff = pl.multiple_of(b * OUT * VP + d0 * VP, 512)
        pltpu.sync_copy(g_v, gt_hbm.at[pl.ds(goff, DPT * VP)])
        return 0

    lax.fori_loop(0, B, per_batch, 0)


def _scatter_call(w2t_flat, edges, counts, gv_pad, emb_weight):
    f = pl.kernel(
        _scatter_body,
        mesh=_mesh,
        compiler_params=_sc_params,
        out_type=(
            jax.ShapeDtypeStruct((B * OUT * VP,), jnp.float32),
            jax.ShapeDtypeStruct((VP, EMB), jnp.float32),
        ),
        scratch_types=[
            pltpu.VMEM((DPT * VP,), jnp.float32),
            pltpu.VMEM((DPT * VP,), jnp.float32),
            pltpu.VMEM((NT * 16,), jnp.int32),
            pltpu.VMEM((CHE,), jnp.int32),
            pltpu.VMEM((CHE,), jnp.int32),
            pltpu.VMEM((CHE,), jnp.int32),
            pltpu.VMEM((CHE,), jnp.int32),
            pltpu.VMEM((CHE,), jnp.int32),
            pltpu.VMEM((CHE,), jnp.int32),
            pltpu.VMEM((CHE,), jnp.int32),
            pltpu.VMEM((CHE,), jnp.int32),
            pltpu.VMEM((CHE,), jnp.int32),
            pltpu.VMEM((RPT,), jnp.int32),
            pltpu.VMEM((CG, EMB), jnp.float32),
            pltpu.VMEM((CG, EMB), jnp.float32),
            pltpu.SMEM((B * NT,), jnp.int32),
            pltpu.SemaphoreType.DMA,
            pltpu.SemaphoreType.DMA,
            pltpu.SemaphoreType.DMA,
            pltpu.SemaphoreType.DMA,
        ],
    )
    return f(w2t_flat, edges, counts, gv_pad, emb_weight)


KBLK = 1280


def _mm_body(a_ref, b_ref, bias_ref, o_ref):
    k = pl.program_id(0)
    acc = jnp.dot(a_ref[...], b_ref[...], preferred_element_type=jnp.float32)

    @pl.when(k == 0)
    def _():
        o_ref[...] = acc + bias_ref[...]

    @pl.when(k != 0)
    def _():
        o_ref[...] += acc


def _mm_call(g2, gemb, bias):
    return pl.pallas_call(
        _mm_body,
        grid=(VP // KBLK,),
        in_specs=[
            pl.BlockSpec((B * OUT, KBLK), lambda k: (0, k)),
            pl.BlockSpec((KBLK, EMB), lambda k: (k, 0)),
            pl.BlockSpec((B * OUT, EMB), lambda k: (0, 0)),
        ],
        out_specs=pl.BlockSpec((B * OUT, EMB), lambda k: (0, 0)),
        out_shape=jax.ShapeDtypeStruct((B * OUT, EMB), jnp.float32),
        compiler_params=pltpu.CompilerParams(
            dimension_semantics=("arbitrary",)),
    )(g2, gemb, bias)


@jax.jit
def kernel(input_ids, emb_weight, adj_indices, adj_values, W_vh, tok_to_graph,
           gv_ids, fc_w, fc_b):
    ids = input_ids.astype(jnp.int32).reshape(-1)
    tok = tok_to_graph.astype(jnp.int32)
    rows = adj_indices[0].astype(jnp.int32)
    cols = adj_indices[1].astype(jnp.int32)
    vals = lax.bitcast_convert_type(adj_values.astype(jnp.float32), jnp.int32)
    wv_pad = jnp.pad(W_vh.astype(jnp.float32), ((0, VP - V), (0, 0)))
    gv_pad = jnp.pad(gv_ids.astype(jnp.int32), (0, VP - V))

    w2t = _w2t_call(fc_w.astype(jnp.float32), wv_pad)
    edges, counts = _compact_call(ids, tok, rows, cols, vals)
    gt, gemb = _scatter_call(w2t.reshape(-1), edges, counts, gv_pad,
                             emb_weight.astype(jnp.float32))

    g2 = gt.reshape(B * OUT, VP)
    bias = jnp.broadcast_to(jnp.tile(fc_b.astype(jnp.float32), B)[:, None],
                            (B * OUT, EMB))
    out = _mm_call(g2, gemb, bias)
    return out.reshape(B, OUT, EMB)


# speedup vs baseline: 1.0521x; 1.0202x over previous
import functools

import jax
import jax.numpy as jnp
from jax import lax
from jax.experimental import pallas as pl
from jax.experimental.pallas import tpu as pltpu
from jax.experimental.pallas import tpu_sc as plsc

B = 4
L = 512
V = 10000
VP = 10240
NNZ = 160000
VOCAB = 30522
EMB = 768
HID = 128
OUT = 128

NT = 32
EPT = NNZ // NT
REGC = 5760
R3 = 3 * REGC
PBUF = 5024
CHE = 640
DPT = OUT // NT
RPT = VP // NT
CG = 16
NCH = RPT // CG

_mesh = plsc.VectorSubcoreMesh(core_axis_name="c", subcore_axis_name="s")
_sc_params = pltpu.CompilerParams(needs_layout_passes=False)


def _wid():
    return lax.axis_index("c") * 16 + lax.axis_index("s")


def _w2t_body(fcw_ref, wv_ref, out_ref):
    out_ref[...] = lax.dot_general(
        fcw_ref[...], wv_ref[...],
        (((1,), (1,)), ((), ())),
        preferred_element_type=jnp.float32)


def _w2t_call(fc_w, wv_pad):
    return pl.pallas_call(
        _w2t_body,
        out_shape=jax.ShapeDtypeStruct((OUT, VP), jnp.float32),
    )(fc_w, wv_pad)


def _compact_body(ids_hbm, tok_hbm, row_hbm, col_hbm, val_hbm,
                  edges_hbm, cnt_hbm,
                  tok_v, ids_v, inset_v, er_v, ec_v, ev_v,
                  rb_a, cb_a, vb_a, rb_b, cb_b, vb_b, cnt_v, sem_w):
    w = _wid()
    iota16 = jnp.arange(16, dtype=jnp.int32)
    zeros16 = jnp.zeros((16,), jnp.int32)
    ones16 = jnp.ones((16,), jnp.int32)

    pltpu.sync_copy(tok_hbm, tok_v)
    pltpu.sync_copy(ids_hbm, ids_v)
    pltpu.sync_copy(row_hbm.at[pl.ds(w * EPT, EPT)], er_v.at[pl.ds(0, EPT)])
    pltpu.sync_copy(col_hbm.at[pl.ds(w * EPT, EPT)], ec_v.at[pl.ds(0, EPT)])
    pltpu.sync_copy(val_hbm.at[pl.ds(w * EPT, EPT)], ev_v.at[pl.ds(0, EPT)])
    er_v[pl.ds(EPT, 16)] = zeros16
    ec_v[pl.ds(EPT, 16)] = zeros16
    ev_v[pl.ds(EPT, 16)] = zeros16

    def zero_step(i, _):
        inset_v[pl.ds(i * 16, 16)] = zeros16
        return 0
    lax.fori_loop(0, (B * V) // 16, zero_step, 0)

    def build_step(i, _):
        b = i // (L // 16)
        s = i % (L // 16)
        idv = ids_v[pl.ds(b * L + s * 16, 16)]
        idv = jnp.where(idv > VOCAB - 1, 0, idv)
        g = plsc.load_gather(tok_v, [idv])
        plsc.store_scatter(inset_v, [g + b * V], ones16)
        return 0
    for i in range(B * (L // 16)):
        build_step(i, 0)

    nsteps = (EPT + 15) // 16
    cvec = zeros16
    for b in range(B):
        bufs = (rb_a, cb_a, vb_a) if b % 2 == 0 else (rb_b, cb_b, vb_b)
        rb_v, cb_v, vb_v = bufs

        def comp_step(i, off, b=b, rb_v=rb_v, cb_v=cb_v, vb_v=vb_v):
            base = i * 16
            valid = (base + iota16) < EPT
            r = er_v[pl.ds(base, 16)]
            c = ec_v[pl.ds(base, 16)]
            v = ev_v[pl.ds(base, 16)]
            inr = plsc.load_gather(inset_v, [r + b * V])
            inc = plsc.load_gather(inset_v, [c + b * V])
            m = ((inr + inc) > 0) & valid
            plsc.store_compressed(rb_v.at[pl.ds(off, 16)], r, mask=m)
            plsc.store_compressed(cb_v.at[pl.ds(off, 16)], c, mask=m)
            plsc.store_compressed(vb_v.at[pl.ds(off, 16)], v, mask=m)
            pc = plsc.all_reduce_population_count(m)
            return off + pc[0]
        if b >= 2:
            for pb in bufs:
                pltpu.make_async_copy(pb, edges_hbm.at[pl.ds(0, PBUF)],
                                      sem_w).wait()
        off = lax.fori_loop(0, nsteps, comp_step, jnp.int32(0))
        cvec = jnp.where(iota16 == b, off, cvec)
        rbase = (b * NT + w) * R3
        for p, pb in enumerate(bufs):
            roff = pl.multiple_of(rbase + p * REGC, 8)
            pltpu.async_copy(pb, edges_hbm.at[pl.ds(roff, PBUF)], sem_w)
    cnt_v[...] = cvec
    for pb in (rb_a, cb_a, vb_a, rb_b, cb_b, vb_b):
        pltpu.make_async_copy(pb, edges_hbm.at[pl.ds(0, PBUF)], sem_w).wait()
    pltpu.sync_copy(cnt_v, cnt_hbm.at[pl.ds(w * 16, 16)])


def _compact_call(ids, tok, rows, cols, vals):
    f = pl.kernel(
        _compact_body,
        mesh=_mesh,
        compiler_params=_sc_params,
        out_type=(
            jax.ShapeDtypeStruct((B * NT * R3,), jnp.int32),
            jax.ShapeDtypeStruct((NT * 16,), jnp.int32),
        ),
        scratch_types=[
            pltpu.VMEM((VOCAB,), jnp.int32),
            pltpu.VMEM((B * L,), jnp.int32),
            pltpu.VMEM((B * V,), jnp.int32),
            pltpu.VMEM((EPT + 16,), jnp.int32),
            pltpu.VMEM((EPT + 16,), jnp.int32),
            pltpu.VMEM((EPT + 16,), jnp.int32),
            pltpu.VMEM((PBUF,), jnp.int32),
            pltpu.VMEM((PBUF,), jnp.int32),
            pltpu.VMEM((PBUF,), jnp.int32),
            pltpu.VMEM((PBUF,), jnp.int32),
            pltpu.VMEM((PBUF,), jnp.int32),
            pltpu.VMEM((PBUF,), jnp.int32),
            pltpu.VMEM((16,), jnp.int32),
            pltpu.SemaphoreType.DMA,
        ],
    )
    return f(ids, tok, rows, cols, vals)


def _scatter_body(w2t_hbm, edges_hbm, cnt_hbm, gv_hbm, emb_hbm,
                  gt_hbm, gemb_hbm, w2_v, g_v, cnt_v,
                  rb_a, cb_a, vb_a, rb_b, cb_b, vb_b, rb_c, cb_c, vb_c,
                  gvi_v, gb_a, gb_b,
                  cnt_s, sem_a, sem_b, gsem_a, gsem_b, ssem_a, ssem_b):
    w = _wid()
    d0 = w * DPT
    iota16 = jnp.arange(16, dtype=jnp.int32)
    zeros16f = jnp.zeros((16,), jnp.float32)
    bufs_a = (rb_a, cb_a, vb_a)
    bufs_b = (rb_b, cb_b, vb_b)
    bufs_c = (rb_c, cb_c, vb_c)

    pltpu.sync_copy(w2t_hbm.at[pl.ds(pl.multiple_of(d0 * VP, 512), DPT * VP)],
                    w2_v)
    pltpu.sync_copy(cnt_hbm, cnt_v)
    pltpu.sync_copy(gv_hbm.at[pl.ds(w * RPT, RPT)], gvi_v)

    def gemb_issue(ch, buf, sem):
        goff = pl.multiple_of(ch * CG, 8)
        pltpu.async_copy(emb_hbm.at[gvi_v.at[pl.ds(goff, CG)]], buf, sem)

    def gemb_drain(buf, sem):
        pltpu.make_async_copy(emb_hbm.at[pl.ds(0, CG)], buf, sem).wait()

    def gemb_astore(ch, buf, sem):
        roff = pl.multiple_of(w * RPT + ch * CG, 8)
        pltpu.async_copy(buf, gemb_hbm.at[pl.ds(roff, CG)], sem)

    def gemb_store_drain(buf, sem):
        pltpu.make_async_copy(buf, gemb_hbm.at[pl.ds(0, CG)], sem).wait()

    def gemb_service(q):
        even = q % 2 == 0

        @pl.when(even & (q >= 2) & (q <= NCH))
        def _():
            gemb_store_drain(gb_a, ssem_a)
            gemb_store_drain(gb_b, ssem_b)

        @pl.when(even & (q <= NCH - 2))
        def _():
            gemb_issue(q, gb_a, gsem_a)
            gemb_issue(q + 1, gb_b, gsem_b)

        @pl.when((~even) & (q <= NCH - 1))
        def _():
            gemb_drain(gb_a, gsem_a)
            gemb_astore(q - 1, gb_a, ssem_a)
            gemb_drain(gb_b, gsem_b)
            gemb_astore(q, gb_b, ssem_b)

    for w2 in range(NT):
        cv = cnt_v[pl.ds(w2 * 16, 16)]
        for b in range(B):
            cnt_s[b * NT + w2] = cv[b]

    def issue(b, rg, bufs, sem, ck=0):
        rg = jnp.minimum(rg, NT - 1)
        rbase = (b * NT + rg) * R3 + ck
        for p in range(3):
            eoff = pl.multiple_of(rbase + p * REGC, 8)
            pltpu.async_copy(edges_hbm.at[pl.ds(eoff, CHE)], bufs[p], sem)

    def drain(bufs, sem):
        for p in range(3):
            pltpu.make_async_copy(edges_hbm.at[pl.ds(0, CHE)], bufs[p],
                                  sem).wait()

    def run_edges(bufs, lim):
        rbuf, cbuf, vbuf = bufs

        def group(base):
            r = rbuf[pl.ds(base, 16)]
            cc = cbuf[pl.ds(base, 16)]
            v = plsc.bitcast(vbuf[pl.ds(base, 16)], jnp.float32)
            for j in range(DPT):
                wv = plsc.load_gather(w2_v, [cc + j * VP])
                plsc.addupdate_scatter(g_v, [r + j * VP], wv * v)

        n64 = lim // 64

        def qstep(s, _):
            t = s * 64
            group(t)
            group(t + 16)
            group(t + 32)
            group(t + 48)
            return 0
        lax.fori_loop(0, n64, qstep, 0)

        n16 = (lim - n64 * 64) // 16
        tb0 = n64 * 64
        for u in range(3):
            @pl.when(n16 >= u + 1)
            def _(u=u):
                group(tb0 + u * 16)

        rem = lim - (lim // 16) * 16

        @pl.when(rem > 0)
        def _():
            base = (lim // 16) * 16
            valid = iota16 < rem
            r = jnp.where(valid, rbuf[pl.ds(base, 16)], 0)
            cc = jnp.where(valid, cbuf[pl.ds(base, 16)], 0)
            v = jnp.where(valid,
                          plsc.bitcast(vbuf[pl.ds(base, 16)], jnp.float32),
                          0.0)
            for j in range(DPT):
                wv = plsc.load_gather(w2_v, [cc + j * VP])
                plsc.addupdate_scatter(g_v, [r + j * VP], wv * v)

    def process(b, rg, bufs):
        n = cnt_s[b * NT + rg]
        run_edges(bufs, jnp.minimum(n, CHE))

        def cond(ck):
            return ck < n

        def obody(ck):
            rbase = (b * NT + rg) * R3 + ck
            for p in range(3):
                eoff = pl.multiple_of(rbase + p * REGC, 8)
                pltpu.sync_copy(edges_hbm.at[pl.ds(eoff, CHE)], bufs_c[p])
            run_edges(bufs_c, jnp.minimum(n - ck, CHE))
            return ck + CHE

        lax.while_loop(cond, obody, jnp.int32(CHE))

    def per_batch(b, _):
        def zero_step(i, _):
            for u in range(8):
                g_v[pl.ds(i * 128 + u * 16, 16)] = zeros16f
            return 0
        lax.fori_loop(0, (DPT * VP) // 128, zero_step, 0)

        issue(b, 0, bufs_a, sem_a)
        issue(b, 1, bufs_b, sem_b)

        def pair(p, _):
            gemb_service(b * (NT // 2) + p)
            drain(bufs_a, sem_a)
            process(b, 2 * p, bufs_a)
            issue(b, 2 * p + 2, bufs_a, sem_a)
            drain(bufs_b, sem_b)
            process(b, 2 * p + 1, bufs_b)
            issue(b, 2 * p + 3, bufs_b, sem_b)
            return 0

        lax.fori_loop(0, NT // 2, pair, 0)
        drain(bufs_a, sem_a)
        drain(bufs_b, sem_b)
        goff = pl.multiple_of(b * OUT * VP + d0 * VP, 512)
        pltpu.sync_copy(g_v, gt_hbm.at[pl.ds(goff, DPT * VP)])
        return 0

    lax.fori_loop(0, B, per_batch, 0)


def _scatter_call(w2t_flat, edges, counts, gv_pad, emb_weight):
    f = pl.kernel(
        _scatter_body,
        mesh=_mesh,
        compiler_params=_sc_params,
        out_type=(
            jax.ShapeDtypeStruct((B * OUT * VP,), jnp.float32),
            jax.ShapeDtypeStruct((VP, EMB), jnp.float32),
        ),
        scratch_types=[
            pltpu.VMEM((DPT * VP,), jnp.float32),
            pltpu.VMEM((DPT * VP,), jnp.float32),
            pltpu.VMEM((NT * 16,), jnp.int32),
            pltpu.VMEM((CHE,), jnp.int32),
            pltpu.VMEM((CHE,), jnp.int32),
            pltpu.VMEM((CHE,), jnp.int32),
            pltpu.VMEM((CHE,), jnp.int32),
            pltpu.VMEM((CHE,), jnp.int32),
            pltpu.VMEM((CHE,), jnp.int32),
            pltpu.VMEM((CHE,), jnp.int32),
            pltpu.VMEM((CHE,), jnp.int32),
            pltpu.VMEM((CHE,), jnp.int32),
            pltpu.VMEM((RPT,), jnp.int32),
            pltpu.VMEM((CG, EMB), jnp.float32),
            pltpu.VMEM((CG, EMB), jnp.float32),
            pltpu.SMEM((B * NT,), jnp.int32),
            pltpu.SemaphoreType.DMA,
            pltpu.SemaphoreType.DMA,
            pltpu.SemaphoreType.DMA,
            pltpu.SemaphoreType.DMA,
            pltpu.SemaphoreType.DMA,
            pltpu.SemaphoreType.DMA,
        ],
    )
    return f(w2t_flat, edges, counts, gv_pad, emb_weight)


KBLK = 1280


def _mm_body(a_ref, b_ref, bias_ref, o_ref):
    k = pl.program_id(0)
    acc = jnp.dot(a_ref[...], b_ref[...], preferred_element_type=jnp.float32)

    @pl.when(k == 0)
    def _():
        o_ref[...] = acc + bias_ref[...]

    @pl.when(k != 0)
    def _():
        o_ref[...] += acc


def _mm_call(g2, gemb, bias):
    return pl.pallas_call(
        _mm_body,
        grid=(VP // KBLK,),
        in_specs=[
            pl.BlockSpec((B * OUT, KBLK), lambda k: (0, k)),
            pl.BlockSpec((KBLK, EMB), lambda k: (k, 0)),
            pl.BlockSpec((B * OUT, EMB), lambda k: (0, 0)),
        ],
        out_specs=pl.BlockSpec((B * OUT, EMB), lambda k: (0, 0)),
        out_shape=jax.ShapeDtypeStruct((B * OUT, EMB), jnp.float32),
        compiler_params=pltpu.CompilerParams(
            dimension_semantics=("arbitrary",)),
    )(g2, gemb, bias)


@jax.jit
def kernel(input_ids, emb_weight, adj_indices, adj_values, W_vh, tok_to_graph,
           gv_ids, fc_w, fc_b):
    ids = input_ids.astype(jnp.int32).reshape(-1)
    tok = tok_to_graph.astype(jnp.int32)
    rows = adj_indices[0].astype(jnp.int32)
    cols = adj_indices[1].astype(jnp.int32)
    vals = lax.bitcast_convert_type(adj_values.astype(jnp.float32), jnp.int32)
    wv_pad = jnp.pad(W_vh.astype(jnp.float32), ((0, VP - V), (0, 0)))
    gv_pad = jnp.pad(gv_ids.astype(jnp.int32), (0, VP - V))

    w2t = _w2t_call(fc_w.astype(jnp.float32), wv_pad)
    edges, counts = _compact_call(ids, tok, rows, cols, vals)
    gt, gemb = _scatter_call(w2t.reshape(-1), edges, counts, gv_pad,
                             emb_weight.astype(jnp.float32))

    g2 = gt.reshape(B * OUT, VP)
    bias = jnp.broadcast_to(jnp.tile(fc_b.astype(jnp.float32), B)[:, None],
                            (B * OUT, EMB))
    out = _mm_call(g2, gemb, bias)
    return out.reshape(B, OUT, EMB)


# speedup vs baseline: 1.0919x; 1.0379x over previous
import functools

import jax
import jax.numpy as jnp
from jax import lax
from jax.experimental import pallas as pl
from jax.experimental.pallas import tpu as pltpu
from jax.experimental.pallas import tpu_sc as plsc

B = 4
L = 512
V = 10000
VP = 10240
NNZ = 160000
VOCAB = 30522
EMB = 768
HID = 128
OUT = 128

NT = 32
EPT = NNZ // NT
REGC = 5760
R3 = 3 * REGC
PBUF = 5024
CHE = 640
DPT = OUT // NT
RPT = VP // NT
CG = 16
NCH = RPT // CG

_mesh = plsc.VectorSubcoreMesh(core_axis_name="c", subcore_axis_name="s")
_sc_params = pltpu.CompilerParams(needs_layout_passes=False)


def _wid():
    return lax.axis_index("c") * 16 + lax.axis_index("s")


def _w2t_body(fcw_ref, wv_ref, out_ref):
    out_ref[...] = lax.dot_general(
        fcw_ref[...], wv_ref[...],
        (((1,), (1,)), ((), ())),
        preferred_element_type=jnp.float32)


def _w2t_call(fc_w, wv_pad):
    return pl.pallas_call(
        _w2t_body,
        out_shape=jax.ShapeDtypeStruct((OUT, VP), jnp.float32),
    )(fc_w, wv_pad)


def _compact_body(ids_hbm, tok_hbm, row_hbm, col_hbm, val_hbm,
                  edges_hbm, cnt_hbm,
                  tok_v, ids_v, inset_v, er_v, ec_v, ev_v,
                  rb_a, cb_a, vb_a, rb_b, cb_b, vb_b, cnt_v, sem_w):
    w = _wid()
    iota16 = jnp.arange(16, dtype=jnp.int32)
    zeros16 = jnp.zeros((16,), jnp.int32)
    ones16 = jnp.ones((16,), jnp.int32)

    pltpu.sync_copy(tok_hbm, tok_v)
    pltpu.sync_copy(ids_hbm, ids_v)
    pltpu.sync_copy(row_hbm.at[pl.ds(w * EPT, EPT)], er_v.at[pl.ds(0, EPT)])
    pltpu.sync_copy(col_hbm.at[pl.ds(w * EPT, EPT)], ec_v.at[pl.ds(0, EPT)])
    pltpu.sync_copy(val_hbm.at[pl.ds(w * EPT, EPT)], ev_v.at[pl.ds(0, EPT)])
    er_v[pl.ds(EPT, 16)] = zeros16
    ec_v[pl.ds(EPT, 16)] = zeros16
    ev_v[pl.ds(EPT, 16)] = zeros16

    def zero_step(i, _):
        for u in range(8):
            inset_v[pl.ds(i * 128 + u * 16, 16)] = zeros16
        return 0
    lax.fori_loop(0, (B * V) // 128, zero_step, 0)
    for u in range((B * V) // 128 * 128, B * V, 16):
        inset_v[pl.ds(u, 16)] = zeros16

    def build_step(i, _):
        b = i // (L // 16)
        s = i % (L // 16)
        idv = ids_v[pl.ds(b * L + s * 16, 16)]
        idv = jnp.where(idv > VOCAB - 1, 0, idv)
        g = plsc.load_gather(tok_v, [idv])
        plsc.store_scatter(inset_v, [g + b * V], ones16)
        return 0
    for i in range(B * (L // 16)):
        build_step(i, 0)

    npairs = EPT // 32
    tail_valid = (npairs * 32 + iota16) < EPT
    true16 = iota16 < 16
    cvec = zeros16
    for b in range(B):
        bufs = (rb_a, cb_a, vb_a) if b % 2 == 0 else (rb_b, cb_b, vb_b)
        rb_v, cb_v, vb_v = bufs

        def cgroup(base, off, valid, b, rb_v, cb_v, vb_v):
            r = er_v[pl.ds(base, 16)]
            c = ec_v[pl.ds(base, 16)]
            v = ev_v[pl.ds(base, 16)]
            inr = plsc.load_gather(inset_v, [r + b * V])
            inc = plsc.load_gather(inset_v, [c + b * V])
            m = ((inr + inc) > 0) & valid
            plsc.store_compressed(rb_v.at[pl.ds(off, 16)], r, mask=m)
            plsc.store_compressed(cb_v.at[pl.ds(off, 16)], c, mask=m)
            plsc.store_compressed(vb_v.at[pl.ds(off, 16)], v, mask=m)
            pc = plsc.all_reduce_population_count(m)
            return off + pc[0]

        def comp_step(i, off, b=b, rb_v=rb_v, cb_v=cb_v, vb_v=vb_v):
            base = i * 32
            off = cgroup(base, off, true16, b, rb_v, cb_v, vb_v)
            off = cgroup(base + 16, off, true16, b, rb_v, cb_v, vb_v)
            return off
        if b >= 2:
            for pb in bufs:
                pltpu.make_async_copy(pb, edges_hbm.at[pl.ds(0, PBUF)],
                                      sem_w).wait()
        off = lax.fori_loop(0, npairs, comp_step, jnp.int32(0))
        off = cgroup(npairs * 32, off, tail_valid, b, rb_v, cb_v, vb_v)
        cvec = jnp.where(iota16 == b, off, cvec)
        rbase = (b * NT + w) * R3
        for p, pb in enumerate(bufs):
            roff = pl.multiple_of(rbase + p * REGC, 8)
            pltpu.async_copy(pb, edges_hbm.at[pl.ds(roff, PBUF)], sem_w)
    cnt_v[...] = cvec
    for pb in (rb_a, cb_a, vb_a, rb_b, cb_b, vb_b):
        pltpu.make_async_copy(pb, edges_hbm.at[pl.ds(0, PBUF)], sem_w).wait()
    pltpu.sync_copy(cnt_v, cnt_hbm.at[pl.ds(w * 16, 16)])


def _compact_call(ids, tok, rows, cols, vals):
    f = pl.kernel(
        _compact_body,
        mesh=_mesh,
        compiler_params=_sc_params,
        out_type=(
            jax.ShapeDtypeStruct((B * NT * R3,), jnp.int32),
            jax.ShapeDtypeStruct((NT * 16,), jnp.int32),
        ),
        scratch_types=[
            pltpu.VMEM((VOCAB,), jnp.int32),
            pltpu.VMEM((B * L,), jnp.int32),
            pltpu.VMEM((B * V,), jnp.int32),
            pltpu.VMEM((EPT + 16,), jnp.int32),
            pltpu.VMEM((EPT + 16,), jnp.int32),
            pltpu.VMEM((EPT + 16,), jnp.int32),
            pltpu.VMEM((PBUF,), jnp.int32),
            pltpu.VMEM((PBUF,), jnp.int32),
            pltpu.VMEM((PBUF,), jnp.int32),
            pltpu.VMEM((PBUF,), jnp.int32),
            pltpu.VMEM((PBUF,), jnp.int32),
            pltpu.VMEM((PBUF,), jnp.int32),
            pltpu.VMEM((16,), jnp.int32),
            pltpu.SemaphoreType.DMA,
        ],
    )
    return f(ids, tok, rows, cols, vals)


def _scatter_body(w2t_hbm, edges_hbm, cnt_hbm, gv_hbm, emb_hbm,
                  gt_hbm, gemb_hbm, w2_v, g_v, cnt_v,
                  rb_a, cb_a, vb_a, rb_b, cb_b, vb_b, rb_c, cb_c, vb_c,
                  gvi_v, gb_a, gb_b,
                  cnt_s, sem_a, sem_b, gsem_a, gsem_b, ssem_a, ssem_b):
    w = _wid()
    d0 = w * DPT
    iota16 = jnp.arange(16, dtype=jnp.int32)
    zeros16f = jnp.zeros((16,), jnp.float32)
    bufs_a = (rb_a, cb_a, vb_a)
    bufs_b = (rb_b, cb_b, vb_b)
    bufs_c = (rb_c, cb_c, vb_c)

    pltpu.sync_copy(w2t_hbm.at[pl.ds(pl.multiple_of(d0 * VP, 512), DPT * VP)],
                    w2_v)
    pltpu.sync_copy(cnt_hbm, cnt_v)
    pltpu.sync_copy(gv_hbm.at[pl.ds(w * RPT, RPT)], gvi_v)

    def gemb_issue(ch, buf, sem):
        goff = pl.multiple_of(ch * CG, 8)
        pltpu.async_copy(emb_hbm.at[gvi_v.at[pl.ds(goff, CG)]], buf, sem)

    def gemb_drain(buf, sem):
        pltpu.make_async_copy(emb_hbm.at[pl.ds(0, CG)], buf, sem).wait()

    def gemb_astore(ch, buf, sem):
        roff = pl.multiple_of(w * RPT + ch * CG, 8)
        pltpu.async_copy(buf, gemb_hbm.at[pl.ds(roff, CG)], sem)

    def gemb_store_drain(buf, sem):
        pltpu.make_async_copy(buf, gemb_hbm.at[pl.ds(0, CG)], sem).wait()

    def gemb_service(q):
        even = q % 2 == 0

        @pl.when(even & (q >= 2) & (q <= NCH))
        def _():
            gemb_store_drain(gb_a, ssem_a)
            gemb_store_drain(gb_b, ssem_b)

        @pl.when(even & (q <= NCH - 2))
        def _():
            gemb_issue(q, gb_a, gsem_a)
            gemb_issue(q + 1, gb_b, gsem_b)

        @pl.when((~even) & (q <= NCH - 1))
        def _():
            gemb_drain(gb_a, gsem_a)
            gemb_astore(q - 1, gb_a, ssem_a)
            gemb_drain(gb_b, gsem_b)
            gemb_astore(q, gb_b, ssem_b)

    for w2 in range(NT):
        cv = cnt_v[pl.ds(w2 * 16, 16)]
        for b in range(B):
            cnt_s[b * NT + w2] = cv[b]

    def issue(b, rg, bufs, sem, ck=0):
        rg = jnp.minimum(rg, NT - 1)
        rbase = (b * NT + rg) * R3 + ck
        for p in range(3):
            eoff = pl.multiple_of(rbase + p * REGC, 8)
            pltpu.async_copy(edges_hbm.at[pl.ds(eoff, CHE)], bufs[p], sem)

    def drain(bufs, sem):
        for p in range(3):
            pltpu.make_async_copy(edges_hbm.at[pl.ds(0, CHE)], bufs[p],
                                  sem).wait()

    def run_edges(bufs, lim):
        rbuf, cbuf, vbuf = bufs

        def group(base):
            r = rbuf[pl.ds(base, 16)]
            cc = cbuf[pl.ds(base, 16)]
            v = plsc.bitcast(vbuf[pl.ds(base, 16)], jnp.float32)
            for j in range(DPT):
                wv = plsc.load_gather(w2_v, [cc + j * VP])
                plsc.addupdate_scatter(g_v, [r + j * VP], wv * v)

        n64 = lim // 64

        def qstep(s, _):
            t = s * 64
            group(t)
            group(t + 16)
            group(t + 32)
            group(t + 48)
            return 0
        lax.fori_loop(0, n64, qstep, 0)

        n16 = (lim - n64 * 64) // 16
        tb0 = n64 * 64
        for u in range(3):
            @pl.when(n16 >= u + 1)
            def _(u=u):
                group(tb0 + u * 16)

        rem = lim - (lim // 16) * 16

        @pl.when(rem > 0)
        def _():
            base = (lim // 16) * 16
            valid = iota16 < rem
            r = jnp.where(valid, rbuf[pl.ds(base, 16)], 0)
            cc = jnp.where(valid, cbuf[pl.ds(base, 16)], 0)
            v = jnp.where(valid,
                          plsc.bitcast(vbuf[pl.ds(base, 16)], jnp.float32),
                          0.0)
            for j in range(DPT):
                wv = plsc.load_gather(w2_v, [cc + j * VP])
                plsc.addupdate_scatter(g_v, [r + j * VP], wv * v)

    def process(b, rg, bufs):
        n = cnt_s[b * NT + rg]
        run_edges(bufs, jnp.minimum(n, CHE))

        def cond(ck):
            return ck < n

        def obody(ck):
            rbase = (b * NT + rg) * R3 + ck
            for p in range(3):
                eoff = pl.multiple_of(rbase + p * REGC, 8)
                pltpu.sync_copy(edges_hbm.at[pl.ds(eoff, CHE)], bufs_c[p])
            run_edges(bufs_c, jnp.minimum(n - ck, CHE))
            return ck + CHE

        lax.while_loop(cond, obody, jnp.int32(CHE))

    def per_batch(b, _):
        def zero_step(i, _):
            for u in range(8):
                g_v[pl.ds(i * 128 + u * 16, 16)] = zeros16f
            return 0
        lax.fori_loop(0, (DPT * VP) // 128, zero_step, 0)

        issue(b, 0, bufs_a, sem_a)
        issue(b, 1, bufs_b, sem_b)

        def pair(p, _):
            gemb_service(b * (NT // 2) + p)
            drain(bufs_a, sem_a)
            process(b, 2 * p, bufs_a)
            issue(b, 2 * p + 2, bufs_a, sem_a)
            drain(bufs_b, sem_b)
            process(b, 2 * p + 1, bufs_b)
            issue(b, 2 * p + 3, bufs_b, sem_b)
            return 0

        lax.fori_loop(0, NT // 2, pair, 0)
        drain(bufs_a, sem_a)
        drain(bufs_b, sem_b)
        goff = pl.multiple_of(b * OUT * VP + d0 * VP, 512)
        pltpu.sync_copy(g_v, gt_hbm.at[pl.ds(goff, DPT * VP)])
        return 0

    lax.fori_loop(0, B, per_batch, 0)


def _scatter_call(w2t_flat, edges, counts, gv_pad, emb_weight):
    f = pl.kernel(
        _scatter_body,
        mesh=_mesh,
        compiler_params=_sc_params,
        out_type=(
            jax.ShapeDtypeStruct((B * OUT * VP,), jnp.float32),
            jax.ShapeDtypeStruct((VP, EMB), jnp.float32),
        ),
        scratch_types=[
            pltpu.VMEM((DPT * VP,), jnp.float32),
            pltpu.VMEM((DPT * VP,), jnp.float32),
            pltpu.VMEM((NT * 16,), jnp.int32),
            pltpu.VMEM((CHE,), jnp.int32),
            pltpu.VMEM((CHE,), jnp.int32),
            pltpu.VMEM((CHE,), jnp.int32),
            pltpu.VMEM((CHE,), jnp.int32),
            pltpu.VMEM((CHE,), jnp.int32),
            pltpu.VMEM((CHE,), jnp.int32),
            pltpu.VMEM((CHE,), jnp.int32),
            pltpu.VMEM((CHE,), jnp.int32),
            pltpu.VMEM((CHE,), jnp.int32),
            pltpu.VMEM((RPT,), jnp.int32),
            pltpu.VMEM((CG, EMB), jnp.float32),
            pltpu.VMEM((CG, EMB), jnp.float32),
            pltpu.SMEM((B * NT,), jnp.int32),
            pltpu.SemaphoreType.DMA,
            pltpu.SemaphoreType.DMA,
            pltpu.SemaphoreType.DMA,
            pltpu.SemaphoreType.DMA,
            pltpu.SemaphoreType.DMA,
            pltpu.SemaphoreType.DMA,
        ],
    )
    return f(w2t_flat, edges, counts, gv_pad, emb_weight)


KBLK = 1280


def _mm_body(a_ref, b_ref, bias_ref, o_ref):
    k = pl.program_id(0)
    acc = jnp.dot(a_ref[...], b_ref[...], preferred_element_type=jnp.float32)

    @pl.when(k == 0)
    def _():
        o_ref[...] = acc + bias_ref[...]

    @pl.when(k != 0)
    def _():
        o_ref[...] += acc


def _mm_call(g2, gemb, bias):
    return pl.pallas_call(
        _mm_body,
        grid=(VP // KBLK,),
        in_specs=[
            pl.BlockSpec((B * OUT, KBLK), lambda k: (0, k)),
            pl.BlockSpec((KBLK, EMB), lambda k: (k, 0)),
            pl.BlockSpec((B * OUT, EMB), lambda k: (0, 0)),
        ],
        out_specs=pl.BlockSpec((B * OUT, EMB), lambda k: (0, 0)),
        out_shape=jax.ShapeDtypeStruct((B * OUT, EMB), jnp.float32),
        compiler_params=pltpu.CompilerParams(
            dimension_semantics=("arbitrary",)),
    )(g2, gemb, bias)


@jax.jit
def kernel(input_ids, emb_weight, adj_indices, adj_values, W_vh, tok_to_graph,
           gv_ids, fc_w, fc_b):
    ids = input_ids.astype(jnp.int32).reshape(-1)
    tok = tok_to_graph.astype(jnp.int32)
    rows = adj_indices[0].astype(jnp.int32)
    cols = adj_indices[1].astype(jnp.int32)
    vals = lax.bitcast_convert_type(adj_values.astype(jnp.float32), jnp.int32)
    wv_pad = jnp.pad(W_vh.astype(jnp.float32), ((0, VP - V), (0, 0)))
    gv_pad = jnp.pad(gv_ids.astype(jnp.int32), (0, VP - V))

    w2t = _w2t_call(fc_w.astype(jnp.float32), wv_pad)
    edges, counts = _compact_call(ids, tok, rows, cols, vals)
    gt, gemb = _scatter_call(w2t.reshape(-1), edges, counts, gv_pad,
                             emb_weight.astype(jnp.float32))

    g2 = gt.reshape(B * OUT, VP)
    bias = jnp.broadcast_to(jnp.tile(fc_b.astype(jnp.float32), B)[:, None],
                            (B * OUT, EMB))
    out = _mm_call(g2, gemb, bias)
    return out.reshape(B, OUT, EMB)
